# Initial kernel scaffold; baseline (speedup 1.0000x reference)
import jax
import jax.numpy as jnp
from jax.experimental import pallas as pl


def kernel(x, edge_index, W1, b1, W2, b2, W3, b3, Wfc, bfc):
    raise NotImplementedError("write your pallas kernel here")


# speedup vs baseline: 6.0936x; 6.0936x over previous
import functools

import jax
import jax.numpy as jnp
from jax import lax
from jax.experimental import pallas as pl
from jax.experimental.pallas import tpu as pltpu
from jax.experimental.pallas import tpu_sc as plsc

N = 100000
E = 1600000
NPAD = 100352
EPAD = 1601536
EB = EPAD // 128
NC, NS = 2, 16
ROWS_PER_TILE = NPAD // NS
BLKN = 512
NBLK = NPAD // BLKN

@functools.lru_cache(maxsize=None)
def _mesh():
    return plsc.VectorSubcoreMesh(core_axis_name="c", subcore_axis_name="s",
                                  num_cores=NC, num_subcores=NS)


def _leaky(z):
    return jnp.where(z >= 0, z, 0.01 * z)


@functools.lru_cache(maxsize=None)
def _make_deg_kernel():
    return functools.partial(
        pl.kernel,
        out_type=jax.ShapeDtypeStruct((NC, NPAD, 16), jnp.float32),
        mesh=_mesh(),
        compiler_params=pltpu.CompilerParams(use_tc_tiling_on_sc=False),
        scratch_types=[
            pltpu.VMEM_SHARED((NPAD, 16), jnp.float32),
            pltpu.VMEM((128,), jnp.int32),
            pltpu.VMEM((128, 16), jnp.float32),
        ],
    )(_deg_body)


def _deg_kernel(dst2d):
    return _make_deg_kernel()(dst2d)


def _deg_body(dst2d, out, acc, idxd, ones_st):
    cid = lax.axis_index("c")
    sid = lax.axis_index("s")
    row0 = sid * ROWS_PER_TILE

    def fill_zero(i, _):
        ones_st[i] = jnp.zeros((16,), jnp.float32)
        return _

    lax.fori_loop(0, 128, fill_zero, None)

    def zero_blk(i, _):
        pltpu.sync_copy(ones_st, acc.at[pl.ds(row0 + i * 128, 128)])
        return _

    lax.fori_loop(0, ROWS_PER_TILE // 128, zero_blk, None)

    def fill_one(i, _):
        ones_st[i] = jnp.ones((16,), jnp.float32)
        return _

    lax.fori_loop(0, 128, fill_one, None)
    plsc.subcore_barrier()

    nblocks = EB // (NC * NS)
    base = (cid * NS + sid) * nblocks

    def blk(i, _):
        pltpu.sync_copy(dst2d.at[base + i], idxd)
        pltpu.sync_copy(ones_st, acc.at[idxd], add=True)
        return _

    lax.fori_loop(0, nblocks, blk, None)
    plsc.subcore_barrier()
    pltpu.sync_copy(acc.at[pl.ds(row0, ROWS_PER_TILE)],
                    out.at[cid, pl.ds(row0, ROWS_PER_TILE)])


@functools.lru_cache(maxsize=None)
def _make_segsum(nch, edge_split):
    nout = NC if edge_split else nch
    nblocks = EB // (NC * NS) if edge_split else EB // NS
    chunks_per_core = 1 if edge_split else nch // NC

    @functools.partial(
        pl.kernel,
        out_type=jax.ShapeDtypeStruct((nout, NPAD, 16), jnp.float32),
        mesh=_mesh(),
        compiler_params=pltpu.CompilerParams(use_tc_tiling_on_sc=False),
        scratch_types=[
            pltpu.VMEM_SHARED((NPAD, 16), jnp.float32),
            pltpu.VMEM((128,), jnp.int32),
            pltpu.VMEM((128,), jnp.int32),
            pltpu.VMEM((128, 16), jnp.float32),
        ],
    )
    def seg(u, src2d, dst2d, out, acc, idxs, idxd, stage):
        cid = lax.axis_index("c")
        sid = lax.axis_index("s")
        row0 = sid * ROWS_PER_TILE

        for k in range(chunks_per_core):
            uch = 0 if edge_split else cid * chunks_per_core + k

            def fill(i, _):
                stage[i] = jnp.zeros((16,), jnp.float32)
                return _

            lax.fori_loop(0, 128, fill, None)

            def zero_blk(i, _):
                pltpu.sync_copy(stage, acc.at[pl.ds(row0 + i * 128, 128)])
                return _

            lax.fori_loop(0, ROWS_PER_TILE // 128, zero_blk, None)
            plsc.subcore_barrier()

            base = ((cid * NS + sid) if edge_split else sid) * nblocks

            def blk(i, _):
                j = base + i
                pltpu.sync_copy(src2d.at[j], idxs)
                pltpu.sync_copy(dst2d.at[j], idxd)
                pltpu.sync_copy(u.at[uch].at[idxs], stage)
                pltpu.sync_copy(stage, acc.at[idxd], add=True)
                return _

            lax.fori_loop(0, nblocks, blk, None)
            plsc.subcore_barrier()

            oidx = cid if edge_split else uch
            pltpu.sync_copy(acc.at[pl.ds(row0, ROWS_PER_TILE)],
                            out.at[oidx, pl.ds(row0, ROWS_PER_TILE)])
            if k + 1 < chunks_per_core:
                plsc.subcore_barrier()

    return seg


def _seg16(u, src2d, dst2d):
    return _make_segsum(1, True)(u, src2d, dst2d)


def _seg64(u, src2d, dst2d):
    return _make_segsum(4, False)(u, src2d, dst2d)


def _prologue_body(xp_ref, degp_ref, dinv_ref, u1_ref):
    i = pl.program_id(0)
    deg = degp_ref[0, :, :1] + degp_ref[1, :, :1] + 1.0
    rows = i * BLKN + lax.broadcasted_iota(jnp.int32, (BLKN, 1), 0)
    dinv = jnp.where(rows < N, lax.rsqrt(deg), 0.0)
    dinv_ref[...] = dinv
    u1_ref[0] = xp_ref[...] * dinv


def _layer_body(Sp_ref, u_ref, dinv_ref, W_ref, b_ref, out_ref, *, nsp, nch_in):
    dinv = dinv_ref[...]
    zs = []
    for c in range(nch_in):
        s = Sp_ref[0, c]
        for p in range(1, nsp):
            s = s + Sp_ref[p, c]
        zs.append((s + u_ref[c]) * dinv)
    zc = jnp.concatenate(zs, axis=1) if nch_in > 1 else zs[0]
    h = jnp.dot(zc, W_ref[...], preferred_element_type=jnp.float32) + b_ref[...]
    h = _leaky(h) * dinv
    for c in range(4):
        out_ref[c] = h[:, c * 16:(c + 1) * 16]


def _final_body(Sp_ref, u_ref, dinv_ref, W3_ref, b3_ref, Wfc_ref, bfc_ref,
                out_ref):
    dinv = dinv_ref[...]
    zs = [(Sp_ref[0, c] + u_ref[c]) * dinv for c in range(4)]
    zc = jnp.concatenate(zs, axis=1)
    h = _leaky(jnp.dot(zc, W3_ref[...], preferred_element_type=jnp.float32)
               + b3_ref[...])
    out_ref[...] = (jnp.dot(h, Wfc_ref[...], preferred_element_type=jnp.float32)
                    + bfc_ref[...])


def _row_spec(shape_prefix, blk_shape):
    nlead = len(shape_prefix)

    def imap(i):
        return tuple([0] * nlead) + (i,) + (0,)

    return pl.BlockSpec(tuple(shape_prefix) + (blk_shape, 16), imap)


def _full_spec(shape):
    return pl.BlockSpec(shape, lambda i: tuple(0 for _ in shape))


_DINV_SPEC = pl.BlockSpec((BLKN, 1), lambda i: (i, 0))


def _prologue(xp, degp):
    return pl.pallas_call(
        _prologue_body,
        grid=(NBLK,),
        in_specs=[
            pl.BlockSpec((BLKN, 16), lambda i: (i, 0)),
            _row_spec((2,), BLKN),
        ],
        out_specs=[
            _DINV_SPEC,
            _row_spec((1,), BLKN),
        ],
        out_shape=[
            jax.ShapeDtypeStruct((NPAD, 1), jnp.float32),
            jax.ShapeDtypeStruct((1, NPAD, 16), jnp.float32),
        ],
    )(xp, degp)


def _layer(Sp, u, dinv, W, b, *, nsp, nch_in):
    body = functools.partial(_layer_body, nsp=nsp, nch_in=nch_in)
    return pl.pallas_call(
        body,
        grid=(NBLK,),
        in_specs=[
            _row_spec((nsp, nch_in), BLKN),
            _row_spec((nch_in,), BLKN),
            _DINV_SPEC,
            _full_spec((16 * nch_in, 64)),
            _full_spec((1, 64)),
        ],
        out_specs=_row_spec((4,), BLKN),
        out_shape=jax.ShapeDtypeStruct((4, NPAD, 16), jnp.float32),
    )(Sp, u, dinv, W, b)


def _final(Sp, u, dinv, W3, b3, Wfcp, bfcp):
    return pl.pallas_call(
        _final_body,
        grid=(NBLK,),
        in_specs=[
            _row_spec((1, 4), BLKN),
            _row_spec((4,), BLKN),
            _DINV_SPEC,
            _full_spec((64, 64)),
            _full_spec((1, 64)),
            _full_spec((64, 16)),
            _full_spec((1, 16)),
        ],
        out_specs=pl.BlockSpec((BLKN, 16), lambda i: (i, 0)),
        out_shape=jax.ShapeDtypeStruct((NPAD, 16), jnp.float32),
    )(Sp, u, dinv, W3, b3, Wfcp, bfcp)


def kernel(x, edge_index, W1, b1, W2, b2, W3, b3, Wfc, bfc):
    f32 = jnp.float32
    src = edge_index[0]
    dst = edge_index[1]
    pad = jnp.full((EPAD - E,), N, jnp.int32)
    src2d = jnp.concatenate([src, pad]).reshape(EB, 128)
    dst2d = jnp.concatenate([dst, pad]).reshape(EB, 128)
    xp = jnp.zeros((NPAD, 16), f32).at[:N, :10].set(x)

    W1p = jnp.zeros((16, 64), f32).at[:10].set(W1)
    Wfcp = jnp.zeros((64, 16), f32).at[:, :10].set(Wfc)
    b1r = b1.reshape(1, 64)
    b2r = b2.reshape(1, 64)
    b3r = b3.reshape(1, 64)
    bfcp = jnp.zeros((1, 16), f32).at[0, :10].set(bfc)

    degp = _deg_kernel(dst2d)
    dinv, u1 = _prologue(xp, degp)
    S1 = _seg16(u1, src2d, dst2d)
    u2 = _layer(S1[:, None], u1, dinv, W1p, b1r, nsp=2, nch_in=1)
    S2 = _seg64(u2, src2d, dst2d)
    u3 = _layer(S2[None], u2, dinv, W2, b2r, nsp=1, nch_in=4)
    S3 = _seg64(u3, src2d, dst2d)
    out = _final(S3[None], u3, dinv, W3, b3r, Wfcp, bfcp)
    return out[:N, :10]


# speedup vs baseline: 10.9255x; 1.7930x over previous
import functools

import jax
import jax.numpy as jnp
from jax import lax
from jax.experimental import pallas as pl
from jax.experimental.pallas import tpu as pltpu
from jax.experimental.pallas import tpu_sc as plsc

N = 100000
E = 1600000
NPAD = 100352
EPAD = 1605632
SBE = 128
EBS = EPAD // SBE
NC, NS = 2, 16
ROWS_PER_TILE = NPAD // NS
BLKN = 512
NBLK = NPAD // BLKN


@functools.lru_cache(maxsize=None)
def _mesh():
    return plsc.VectorSubcoreMesh(core_axis_name="c", subcore_axis_name="s",
                                  num_cores=NC, num_subcores=NS)


def _leaky(z):
    return jnp.where(z >= 0, z, 0.01 * z)


def _zero_acc(acc, zsrc, row0):
    def fill(i, _):
        zsrc[i] = jnp.zeros((16,), jnp.float32)
        return _

    lax.fori_loop(0, 128, fill, None)

    def zero_blk(i, _):
        pltpu.sync_copy(zsrc.at[pl.ds(0, 128)],
                        acc.at[pl.ds(row0 + i * 128, 128)])
        return _

    lax.fori_loop(0, ROWS_PER_TILE // 128, zero_blk, None)


@functools.lru_cache(maxsize=None)
def _make_deg_kernel():
    nsb = EBS // (NC * NS)

    def body(dst2d, out, acc, i3d, ones_st, ssem, isem):
        cid = lax.axis_index("c")
        sid = lax.axis_index("s")
        row0 = sid * ROWS_PER_TILE

        _zero_acc(acc, ones_st, row0)

        def fill_one(i, _):
            ones_st[i] = jnp.ones((16,), jnp.float32)
            return _

        lax.fori_loop(0, SBE, fill_one, None)
        plsc.subcore_barrier()

        base = (cid * NS + sid) * nsb

        def idx_load(g, m, sem=None):
            srcslc = dst2d.at[base + g]
            if sem is None:
                pltpu.sync_copy(srcslc, i3d.at[m])
            else:
                pltpu.async_copy(srcslc, i3d.at[m], sem)

        def drain_scatter():
            pltpu.make_async_copy(ones_st, acc.at[i3d.at[0]], ssem).wait()

        def drain_idx():
            pltpu.make_async_copy(dst2d.at[0],
                                  i3d.at[0], isem).wait()

        idx_load(0, 0)
        idx_load(1, 1)

        def it(g, _):
            m = lax.rem(g, 3)
            ml = lax.rem(g + 2, 3)

            @pl.when(g > 0)
            def _ws():
                drain_scatter()

            @pl.when(g + 2 < nsb)
            def _l():
                idx_load(g + 2, ml, isem)

            @pl.when(jnp.logical_and(g > 0, g + 1 < nsb))
            def _di():
                drain_idx()

            pltpu.async_copy(ones_st, acc.at[i3d.at[m]], ssem, add=True)
            return _

        lax.fori_loop(0, nsb, it, None)
        drain_scatter()
        plsc.subcore_barrier()
        pltpu.sync_copy(acc.at[pl.ds(row0, ROWS_PER_TILE)],
                        out.at[cid, pl.ds(row0, ROWS_PER_TILE)])

    return functools.partial(
        pl.kernel,
        out_type=jax.ShapeDtypeStruct((NC, NPAD, 16), jnp.float32),
        mesh=_mesh(),
        compiler_params=pltpu.CompilerParams(use_tc_tiling_on_sc=False),
        scratch_types=[
            pltpu.VMEM_SHARED((NPAD, 16), jnp.float32),
            pltpu.VMEM((3, SBE), jnp.int32),
            pltpu.VMEM((SBE, 16), jnp.float32),
            pltpu.SemaphoreType.DMA,
            pltpu.SemaphoreType.DMA,
        ],
    )(body)


def _deg_kernel(dst2d):
    return _make_deg_kernel()(dst2d)


@functools.lru_cache(maxsize=None)
def _make_segsum(nch, edge_split):
    nout = NC if edge_split else nch
    nsb = EBS // (NC * NS) if edge_split else EBS // NS
    chunks_per_core = 1 if edge_split else nch // NC

    def seg(u, src2d, dst2d, out, acc, i3s, i3d, st2, gsem, ssem, isem):
        cid = lax.axis_index("c")
        sid = lax.axis_index("s")
        row0 = sid * ROWS_PER_TILE
        base = ((cid * NS + sid) if edge_split else sid) * nsb

        for k in range(chunks_per_core):
            if edge_split:
                uref = u.at[0]
            else:
                uref = u.at[cid * chunks_per_core + k]

            _zero_acc(acc, st2.at[0], row0)
            plsc.subcore_barrier()

            def idx_load(g, m, sem=None):
                s_slc = src2d.at[base + g]
                d_slc = dst2d.at[base + g]
                if sem is None:
                    pltpu.sync_copy(s_slc, i3s.at[m])
                    pltpu.sync_copy(d_slc, i3d.at[m])
                else:
                    pltpu.async_copy(s_slc, i3s.at[m], sem)
                    pltpu.async_copy(d_slc, i3d.at[m], sem)

            def drain_gather(uref=uref):
                pltpu.make_async_copy(uref.at[i3s.at[0]], st2.at[0],
                                      gsem).wait()

            def drain_scatter():
                pltpu.make_async_copy(st2.at[0], acc.at[i3d.at[0]],
                                      ssem).wait()

            def drain_idx():
                pltpu.make_async_copy(src2d.at[0],
                                      i3s.at[0], isem).wait()
                pltpu.make_async_copy(src2d.at[0],
                                      i3d.at[0], isem).wait()

            idx_load(0, 0)
            idx_load(1, 1)
            pltpu.async_copy(uref.at[i3s.at[0]], st2.at[0], gsem)

            def it(g, _, uref=uref):
                p = lax.rem(g, 2)
                pn = lax.rem(g + 1, 2)
                m = lax.rem(g, 3)
                mn = lax.rem(g + 1, 3)
                ml = lax.rem(g + 2, 3)

                drain_gather()

                @pl.when(g > 0)
                def _ws():
                    drain_scatter()

                @pl.when(g + 2 < nsb)
                def _l():
                    idx_load(g + 2, ml, isem)

                @pl.when(g + 1 < nsb)
                def _g():
                    @pl.when(g > 0)
                    def _di():
                        drain_idx()
                    pltpu.async_copy(uref.at[i3s.at[mn]], st2.at[pn], gsem)

                pltpu.async_copy(st2.at[p], acc.at[i3d.at[m]], ssem, add=True)
                return _

            lax.fori_loop(0, nsb, it, None)
            drain_scatter()
            plsc.subcore_barrier()

            oidx = cid if edge_split else cid * chunks_per_core + k
            pltpu.sync_copy(acc.at[pl.ds(row0, ROWS_PER_TILE)],
                            out.at[oidx, pl.ds(row0, ROWS_PER_TILE)])
            if k + 1 < chunks_per_core:
                plsc.subcore_barrier()

    return functools.partial(
        pl.kernel,
        out_type=jax.ShapeDtypeStruct((nout, NPAD, 16), jnp.float32),
        mesh=_mesh(),
        compiler_params=pltpu.CompilerParams(use_tc_tiling_on_sc=False),
        scratch_types=[
            pltpu.VMEM_SHARED((NPAD, 16), jnp.float32),
            pltpu.VMEM((3, SBE), jnp.int32),
            pltpu.VMEM((3, SBE), jnp.int32),
            pltpu.VMEM((2, SBE, 16), jnp.float32),
            pltpu.SemaphoreType.DMA,
            pltpu.SemaphoreType.DMA,
            pltpu.SemaphoreType.DMA,
        ],
    )(seg)


def _seg16(u, src2d, dst2d):
    return _make_segsum(1, True)(u, src2d, dst2d)


def _seg64(u, src2d, dst2d):
    return _make_segsum(4, False)(u, src2d, dst2d)


def _prologue_body(xp_ref, degp_ref, dinv_ref, u1_ref):
    i = pl.program_id(0)
    deg = degp_ref[0, :, :1] + degp_ref[1, :, :1] + 1.0
    rows = i * BLKN + lax.broadcasted_iota(jnp.int32, (BLKN, 1), 0)
    dinv = jnp.where(rows < N, lax.rsqrt(deg), 0.0)
    dinv_ref[...] = dinv
    u1_ref[0] = xp_ref[...] * dinv


def _layer_body(Sp_ref, u_ref, dinv_ref, W_ref, b_ref, out_ref, *, nsp, nch_in):
    dinv = dinv_ref[...]
    zs = []
    for c in range(nch_in):
        s = Sp_ref[0, c]
        for p in range(1, nsp):
            s = s + Sp_ref[p, c]
        zs.append((s + u_ref[c]) * dinv)
    zc = jnp.concatenate(zs, axis=1) if nch_in > 1 else zs[0]
    h = jnp.dot(zc, W_ref[...], preferred_element_type=jnp.float32) + b_ref[...]
    h = _leaky(h) * dinv
    for c in range(4):
        out_ref[c] = h[:, c * 16:(c + 1) * 16]


def _final_body(Sp_ref, u_ref, dinv_ref, W3_ref, b3_ref, Wfc_ref, bfc_ref,
                out_ref):
    dinv = dinv_ref[...]
    zs = [(Sp_ref[0, c] + u_ref[c]) * dinv for c in range(4)]
    zc = jnp.concatenate(zs, axis=1)
    h = _leaky(jnp.dot(zc, W3_ref[...], preferred_element_type=jnp.float32)
               + b3_ref[...])
    out_ref[...] = (jnp.dot(h, Wfc_ref[...], preferred_element_type=jnp.float32)
                    + bfc_ref[...])


def _row_spec(shape_prefix, blk_shape):
    nlead = len(shape_prefix)

    def imap(i):
        return tuple([0] * nlead) + (i, 0)

    return pl.BlockSpec(tuple(shape_prefix) + (blk_shape, 16), imap)


def _full_spec(shape):
    return pl.BlockSpec(shape, lambda i: tuple(0 for _ in shape))


_DINV_SPEC = pl.BlockSpec((BLKN, 1), lambda i: (i, 0))


def _prologue(xp, degp):
    return pl.pallas_call(
        _prologue_body,
        grid=(NBLK,),
        in_specs=[
            pl.BlockSpec((BLKN, 16), lambda i: (i, 0)),
            _row_spec((2,), BLKN),
        ],
        out_specs=[
            _DINV_SPEC,
            _row_spec((1,), BLKN),
        ],
        out_shape=[
            jax.ShapeDtypeStruct((NPAD, 1), jnp.float32),
            jax.ShapeDtypeStruct((1, NPAD, 16), jnp.float32),
        ],
    )(xp, degp)


def _layer(Sp, u, dinv, W, b, *, nsp, nch_in):
    body = functools.partial(_layer_body, nsp=nsp, nch_in=nch_in)
    return pl.pallas_call(
        body,
        grid=(NBLK,),
        in_specs=[
            _row_spec((nsp, nch_in), BLKN),
            _row_spec((nch_in,), BLKN),
            _DINV_SPEC,
            _full_spec((16 * nch_in, 64)),
            _full_spec((1, 64)),
        ],
        out_specs=_row_spec((4,), BLKN),
        out_shape=jax.ShapeDtypeStruct((4, NPAD, 16), jnp.float32),
    )(Sp, u, dinv, W, b)


def _final(Sp, u, dinv, W3, b3, Wfcp, bfcp):
    return pl.pallas_call(
        _final_body,
        grid=(NBLK,),
        in_specs=[
            _row_spec((1, 4), BLKN),
            _row_spec((4,), BLKN),
            _DINV_SPEC,
            _full_spec((64, 64)),
            _full_spec((1, 64)),
            _full_spec((64, 16)),
            _full_spec((1, 16)),
        ],
        out_specs=pl.BlockSpec((BLKN, 16), lambda i: (i, 0)),
        out_shape=jax.ShapeDtypeStruct((NPAD, 16), jnp.float32),
    )(Sp, u, dinv, W3, b3, Wfcp, bfcp)


def kernel(x, edge_index, W1, b1, W2, b2, W3, b3, Wfc, bfc):
    f32 = jnp.float32
    src = edge_index[0]
    dst = edge_index[1]
    pad = jnp.full((EPAD - E,), N, jnp.int32)
    src2d = jnp.concatenate([src, pad]).reshape(EBS, SBE)
    dst2d = jnp.concatenate([dst, pad]).reshape(EBS, SBE)
    xp = jnp.zeros((NPAD, 16), f32).at[:N, :10].set(x)

    W1p = jnp.zeros((16, 64), f32).at[:10].set(W1)
    Wfcp = jnp.zeros((64, 16), f32).at[:, :10].set(Wfc)
    b1r = b1.reshape(1, 64)
    b2r = b2.reshape(1, 64)
    b3r = b3.reshape(1, 64)
    bfcp = jnp.zeros((1, 16), f32).at[0, :10].set(bfc)

    degp = _deg_kernel(dst2d)
    dinv, u1 = _prologue(xp, degp)
    S1 = _seg16(u1, src2d, dst2d)
    u2 = _layer(S1[:, None], u1, dinv, W1p, b1r, nsp=2, nch_in=1)
    S2 = _seg64(u2, src2d, dst2d)
    u3 = _layer(S2[None], u2, dinv, W2, b2r, nsp=1, nch_in=4)
    S3 = _seg64(u3, src2d, dst2d)
    out = _final(S3[None], u3, dinv, W3, b3r, Wfcp, bfcp)
    return out[:N, :10]


# speedup vs baseline: 11.8508x; 1.0847x over previous
import functools

import jax
import jax.numpy as jnp
from jax import lax
from jax.experimental import pallas as pl
from jax.experimental.pallas import tpu as pltpu
from jax.experimental.pallas import tpu_sc as plsc

N = 100000
E = 1600000
NPAD = 100352
EPAD = 1605632
SBE = 128
EBS = EPAD // SBE
G = 4
NC, NS = 2, 16
ROWS_PER_TILE = NPAD // NS
BLKN = 1024
NBLK = NPAD // BLKN


@functools.lru_cache(maxsize=None)
def _mesh():
    return plsc.VectorSubcoreMesh(core_axis_name="c", subcore_axis_name="s",
                                  num_cores=NC, num_subcores=NS)


def _leaky(z):
    return jnp.where(z >= 0, z, 0.01 * z)


def _zero_acc(acc, zsrc, row0):
    def fill(i, _):
        zsrc[i] = jnp.zeros((16,), jnp.float32)
        return _

    lax.fori_loop(0, 128, fill, None)

    def zero_blk(i, _):
        pltpu.sync_copy(zsrc.at[pl.ds(0, 128)],
                        acc.at[pl.ds(row0 + i * 128, 128)])
        return _

    lax.fori_loop(0, ROWS_PER_TILE // 128, zero_blk, None)


@functools.lru_cache(maxsize=None)
def _make_deg_kernel():
    nsb = EBS // (NC * NS)

    def body(dst2d, out, acc, i3d, ones_st, ssem, isem):
        cid = lax.axis_index("c")
        sid = lax.axis_index("s")
        row0 = sid * ROWS_PER_TILE

        _zero_acc(acc, ones_st, row0)

        def fill_one(i, _):
            ones_st[i] = jnp.ones((16,), jnp.float32)
            return _

        lax.fori_loop(0, SBE, fill_one, None)
        plsc.subcore_barrier()

        base = (cid * NS + sid) * nsb
        ng = nsb // G

        def idx_load(g, m, sem=None):
            srcslc = dst2d.at[pl.ds(base + g * G, G)]
            if sem is None:
                pltpu.sync_copy(srcslc, i3d.at[m])
            else:
                pltpu.async_copy(srcslc, i3d.at[m], sem)

        def drain_scatter():
            pltpu.make_async_copy(ones_st, acc.at[i3d.at[0, 0]], ssem).wait()

        def drain_idx():
            pltpu.make_async_copy(dst2d.at[pl.ds(0, G)],
                                  i3d.at[0], isem).wait()

        idx_load(0, 0)
        idx_load(1, 1)

        def it(g, _):
            m = lax.rem(g, 3)
            ml = lax.rem(g + 2, 3)

            @pl.when(g > 0)
            def _ws():
                for _j in range(G):
                    drain_scatter()

            @pl.when(g + 2 < ng)
            def _l():
                idx_load(g + 2, ml, isem)

            @pl.when(g >= 2)
            def _di():
                drain_idx()

            for j in range(G):
                pltpu.async_copy(ones_st, acc.at[i3d.at[m, j]], ssem,
                                 add=True)
            return _

        lax.fori_loop(0, ng, it, None)
        for _j in range(G):
            drain_scatter()
        plsc.subcore_barrier()
        pltpu.sync_copy(acc.at[pl.ds(row0, ROWS_PER_TILE)],
                        out.at[cid, pl.ds(row0, ROWS_PER_TILE)])

    return functools.partial(
        pl.kernel,
        out_type=jax.ShapeDtypeStruct((NC, NPAD, 16), jnp.float32),
        mesh=_mesh(),
        compiler_params=pltpu.CompilerParams(use_tc_tiling_on_sc=False),
        scratch_types=[
            pltpu.VMEM_SHARED((NPAD, 16), jnp.float32),
            pltpu.VMEM((3, G, SBE), jnp.int32),
            pltpu.VMEM((SBE, 16), jnp.float32),
            pltpu.SemaphoreType.DMA,
            pltpu.SemaphoreType.DMA,
        ],
    )(body)


def _deg_kernel(dst2d):
    return _make_deg_kernel()(dst2d)


@functools.lru_cache(maxsize=None)
def _make_segsum(nch, edge_split):
    nout = NC if edge_split else nch
    nsb = EBS // (NC * NS) if edge_split else EBS // NS
    chunks_per_core = 1 if edge_split else nch // NC

    def seg(u, src2d, dst2d, out, acc, i3s, i3d, st2, gsem, ssem, isem):
        cid = lax.axis_index("c")
        sid = lax.axis_index("s")
        row0 = sid * ROWS_PER_TILE
        base = ((cid * NS + sid) if edge_split else sid) * nsb

        ng = nsb // G

        for k in range(chunks_per_core):
            if edge_split:
                uref = u.at[0]
            else:
                uref = u.at[cid * chunks_per_core + k]

            _zero_acc(acc, st2.at[0], row0)
            plsc.subcore_barrier()

            def idx_load(g, m, sem=None):
                s_slc = src2d.at[pl.ds((base + g * G), G)]
                d_slc = dst2d.at[pl.ds((base + g * G), G)]
                if sem is None:
                    pltpu.sync_copy(s_slc, i3s.at[m])
                    pltpu.sync_copy(d_slc, i3d.at[m])
                else:
                    pltpu.async_copy(s_slc, i3s.at[m], sem)
                    pltpu.async_copy(d_slc, i3d.at[m], sem)

            def drain_gather(uref=uref):
                pltpu.make_async_copy(uref.at[i3s.at[0, 0]], st2.at[0],
                                      gsem).wait()

            def drain_scatter():
                pltpu.make_async_copy(st2.at[0], acc.at[i3d.at[0, 0]],
                                      ssem).wait()

            def drain_idx():
                pltpu.make_async_copy(src2d.at[pl.ds(0, G)],
                                      i3s.at[0], isem).wait()
                pltpu.make_async_copy(src2d.at[pl.ds(0, G)],
                                      i3d.at[0], isem).wait()

            idx_load(0, 0)
            idx_load(1, 1)
            pltpu.async_copy(uref.at[i3s.at[0, 0]], st2.at[0], gsem)

            def it(g, _, uref=uref):
                m = lax.rem(g, 3)
                mn = lax.rem(g + 1, 3)
                ml = lax.rem(g + 2, 3)

                for j in range(G):
                    drain_gather()

                    if j == 0:
                        @pl.when(g > 0)
                        def _ws():
                            drain_scatter()

                        @pl.when(g + 2 < ng)
                        def _l():
                            idx_load(g + 2, ml, isem)
                    else:
                        drain_scatter()

                    if j < G - 1:
                        pltpu.async_copy(uref.at[i3s.at[m, j + 1]],
                                         st2.at[(j + 1) % 2], gsem)
                    else:
                        @pl.when(g + 1 < ng)
                        def _g():
                            @pl.when(g > 0)
                            def _di():
                                drain_idx()
                            pltpu.async_copy(uref.at[i3s.at[mn, 0]],
                                             st2.at[0], gsem)

                    pltpu.async_copy(st2.at[j % 2], acc.at[i3d.at[m, j]],
                                     ssem, add=True)
                return _

            lax.fori_loop(0, ng, it, None)
            drain_scatter()
            plsc.subcore_barrier()

            oidx = cid if edge_split else cid * chunks_per_core + k
            pltpu.sync_copy(acc.at[pl.ds(row0, ROWS_PER_TILE)],
                            out.at[oidx, pl.ds(row0, ROWS_PER_TILE)])
            if k + 1 < chunks_per_core:
                plsc.subcore_barrier()

    return functools.partial(
        pl.kernel,
        out_type=jax.ShapeDtypeStruct((nout, NPAD, 16), jnp.float32),
        mesh=_mesh(),
        compiler_params=pltpu.CompilerParams(use_tc_tiling_on_sc=False),
        scratch_types=[
            pltpu.VMEM_SHARED((NPAD, 16), jnp.float32),
            pltpu.VMEM((3, G, SBE), jnp.int32),
            pltpu.VMEM((3, G, SBE), jnp.int32),
            pltpu.VMEM((2, SBE, 16), jnp.float32),
            pltpu.SemaphoreType.DMA,
            pltpu.SemaphoreType.DMA,
            pltpu.SemaphoreType.DMA,
        ],
    )(seg)


def _seg16(u, src2d, dst2d):
    return _make_segsum(1, True)(u, src2d, dst2d)


def _seg64(u, src2d, dst2d):
    return _make_segsum(4, False)(u, src2d, dst2d)


def _prologue_body(xp_ref, degp_ref, dinv_ref, u1_ref):
    i = pl.program_id(0)
    deg = degp_ref[0, :, :1] + degp_ref[1, :, :1] + 1.0
    rows = i * BLKN + lax.broadcasted_iota(jnp.int32, (BLKN, 1), 0)
    dinv = jnp.where(rows < N, lax.rsqrt(deg), 0.0)
    dinv_ref[...] = dinv
    u1_ref[0] = xp_ref[...] * dinv


def _layer_body(Sp_ref, u_ref, dinv_ref, W_ref, b_ref, out_ref, *, nsp, nch_in):
    dinv = dinv_ref[...]
    zs = []
    for c in range(nch_in):
        s = Sp_ref[0, c]
        for p in range(1, nsp):
            s = s + Sp_ref[p, c]
        zs.append((s + u_ref[c]) * dinv)
    zc = jnp.concatenate(zs, axis=1) if nch_in > 1 else zs[0]
    h = jnp.dot(zc, W_ref[...], preferred_element_type=jnp.float32) + b_ref[...]
    h = _leaky(h) * dinv
    for c in range(4):
        out_ref[c] = h[:, c * 16:(c + 1) * 16]


def _final_body(Sp_ref, u_ref, dinv_ref, W3_ref, b3_ref, Wfc_ref, bfc_ref,
                out_ref):
    dinv = dinv_ref[...]
    zs = [(Sp_ref[0, c] + u_ref[c]) * dinv for c in range(4)]
    zc = jnp.concatenate(zs, axis=1)
    h = _leaky(jnp.dot(zc, W3_ref[...], preferred_element_type=jnp.float32)
               + b3_ref[...])
    out_ref[...] = (jnp.dot(h, Wfc_ref[...], preferred_element_type=jnp.float32)
                    + bfc_ref[...])


def _row_spec(shape_prefix, blk_shape):
    nlead = len(shape_prefix)

    def imap(i):
        return tuple([0] * nlead) + (i, 0)

    return pl.BlockSpec(tuple(shape_prefix) + (blk_shape, 16), imap)


def _full_spec(shape):
    return pl.BlockSpec(shape, lambda i: tuple(0 for _ in shape))


_DINV_SPEC = pl.BlockSpec((BLKN, 1), lambda i: (i, 0))


def _prologue(xp, degp):
    return pl.pallas_call(
        _prologue_body,
        grid=(NBLK,),
        in_specs=[
            pl.BlockSpec((BLKN, 16), lambda i: (i, 0)),
            _row_spec((2,), BLKN),
        ],
        out_specs=[
            _DINV_SPEC,
            _row_spec((1,), BLKN),
        ],
        out_shape=[
            jax.ShapeDtypeStruct((NPAD, 1), jnp.float32),
            jax.ShapeDtypeStruct((1, NPAD, 16), jnp.float32),
        ],
    )(xp, degp)


def _layer(Sp, u, dinv, W, b, *, nsp, nch_in):
    body = functools.partial(_layer_body, nsp=nsp, nch_in=nch_in)
    return pl.pallas_call(
        body,
        grid=(NBLK,),
        in_specs=[
            _row_spec((nsp, nch_in), BLKN),
            _row_spec((nch_in,), BLKN),
            _DINV_SPEC,
            _full_spec((16 * nch_in, 64)),
            _full_spec((1, 64)),
        ],
        out_specs=_row_spec((4,), BLKN),
        out_shape=jax.ShapeDtypeStruct((4, NPAD, 16), jnp.float32),
    )(Sp, u, dinv, W, b)


def _final(Sp, u, dinv, W3, b3, Wfcp, bfcp):
    return pl.pallas_call(
        _final_body,
        grid=(NBLK,),
        in_specs=[
            _row_spec((1, 4), BLKN),
            _row_spec((4,), BLKN),
            _DINV_SPEC,
            _full_spec((64, 64)),
            _full_spec((1, 64)),
            _full_spec((64, 16)),
            _full_spec((1, 16)),
        ],
        out_specs=pl.BlockSpec((BLKN, 16), lambda i: (i, 0)),
        out_shape=jax.ShapeDtypeStruct((NPAD, 16), jnp.float32),
    )(Sp, u, dinv, W3, b3, Wfcp, bfcp)


def kernel(x, edge_index, W1, b1, W2, b2, W3, b3, Wfc, bfc):
    f32 = jnp.float32
    src = edge_index[0]
    dst = edge_index[1]
    pad = jnp.full((EPAD - E,), N, jnp.int32)
    src2d = jnp.concatenate([src, pad]).reshape(EBS, SBE)
    dst2d = jnp.concatenate([dst, pad]).reshape(EBS, SBE)
    xp = jnp.zeros((NPAD, 16), f32).at[:N, :10].set(x)

    W1p = jnp.zeros((16, 64), f32).at[:10].set(W1)
    Wfcp = jnp.zeros((64, 16), f32).at[:, :10].set(Wfc)
    b1r = b1.reshape(1, 64)
    b2r = b2.reshape(1, 64)
    b3r = b3.reshape(1, 64)
    bfcp = jnp.zeros((1, 16), f32).at[0, :10].set(bfc)

    degp = _deg_kernel(dst2d)
    dinv, u1 = _prologue(xp, degp)
    S1 = _seg16(u1, src2d, dst2d)
    u2 = _layer(S1[:, None], u1, dinv, W1p, b1r, nsp=2, nch_in=1)
    S2 = _seg64(u2, src2d, dst2d)
    u3 = _layer(S2[None], u2, dinv, W2, b2r, nsp=1, nch_in=4)
    S3 = _seg64(u3, src2d, dst2d)
    out = _final(S3[None], u3, dinv, W3, b3r, Wfcp, bfcp)
    return out[:N, :10]


# speedup vs baseline: 17.7884x; 1.5010x over previous
import functools

import jax
import jax.numpy as jnp
from jax import lax
from jax.experimental import pallas as pl
from jax.experimental.pallas import tpu as pltpu
from jax.experimental.pallas import tpu_sc as plsc

N = 100000
E = 1600000
NPAD = 100352
EPAD = 1605632
SBE = 128
EBS = EPAD // SBE
G = 4
NC, NS = 2, 16
ROWS_PER_TILE = NPAD // NS
BLKN = 1024
NBLK = NPAD // BLKN


@functools.lru_cache(maxsize=None)
def _mesh():
    return plsc.VectorSubcoreMesh(core_axis_name="c", subcore_axis_name="s",
                                  num_cores=NC, num_subcores=NS)


def _leaky(z):
    return jnp.where(z >= 0, z, 0.01 * z)


def _zero_acc(acc, zsrc, row0):
    def fill(i, _):
        zsrc[i] = jnp.zeros((16,), jnp.float32)
        return _

    lax.fori_loop(0, 128, fill, None)

    def zero_blk(i, _):
        pltpu.sync_copy(zsrc.at[pl.ds(0, 128)],
                        acc.at[pl.ds(row0 + i * 128, 128)])
        return _

    lax.fori_loop(0, ROWS_PER_TILE // 128, zero_blk, None)


@functools.lru_cache(maxsize=None)
def _make_deg_kernel():
    nsb = EBS // (NC * NS)

    def body(dst2d, out, acc, i3d, ones_st, ssem, isem):
        cid = lax.axis_index("c")
        sid = lax.axis_index("s")
        row0 = sid * ROWS_PER_TILE

        _zero_acc(acc, ones_st, row0)

        def fill_one(i, _):
            ones_st[i] = jnp.ones((16,), jnp.float32)
            return _

        lax.fori_loop(0, SBE, fill_one, None)
        plsc.subcore_barrier()

        base = (cid * NS + sid) * nsb
        ng = nsb // G

        def idx_load(g, m, sem=None):
            srcslc = dst2d.at[pl.ds(base + g * G, G)]
            if sem is None:
                pltpu.sync_copy(srcslc, i3d.at[m])
            else:
                pltpu.async_copy(srcslc, i3d.at[m], sem)

        def drain_scatter():
            pltpu.make_async_copy(ones_st, acc.at[i3d.at[0, 0]], ssem).wait()

        def drain_idx():
            pltpu.make_async_copy(dst2d.at[pl.ds(0, G)],
                                  i3d.at[0], isem).wait()

        idx_load(0, 0)
        idx_load(1, 1)

        def it(g, _):
            m = lax.rem(g, 3)
            ml = lax.rem(g + 2, 3)

            @pl.when(g > 0)
            def _ws():
                for _j in range(G):
                    drain_scatter()

            @pl.when(g + 2 < ng)
            def _l():
                idx_load(g + 2, ml, isem)

            @pl.when(g >= 2)
            def _di():
                drain_idx()

            for j in range(G):
                pltpu.async_copy(ones_st, acc.at[i3d.at[m, j]], ssem,
                                 add=True)
            return _

        lax.fori_loop(0, ng, it, None)
        for _j in range(G):
            drain_scatter()
        plsc.subcore_barrier()
        pltpu.sync_copy(acc.at[pl.ds(row0, ROWS_PER_TILE)],
                        out.at[cid, pl.ds(row0, ROWS_PER_TILE)])

    return functools.partial(
        pl.kernel,
        out_type=jax.ShapeDtypeStruct((NC, NPAD, 16), jnp.float32),
        mesh=_mesh(),
        compiler_params=pltpu.CompilerParams(use_tc_tiling_on_sc=False),
        scratch_types=[
            pltpu.VMEM_SHARED((NPAD, 16), jnp.float32),
            pltpu.VMEM((3, G, SBE), jnp.int32),
            pltpu.VMEM((SBE, 16), jnp.float32),
            pltpu.SemaphoreType.DMA,
            pltpu.SemaphoreType.DMA,
        ],
    )(body)


def _deg_kernel(dst2d):
    return _make_deg_kernel()(dst2d)


@functools.lru_cache(maxsize=None)
def _make_segsum(nch, edge_split):
    nout = NC if edge_split else nch
    nsb = EBS // (NC * NS) if edge_split else EBS // NS
    chunks_per_core = 1 if edge_split else nch // NC

    def seg(u, src2d, dst2d, out, acc, i3s, i3d, st2, gsem, ssem, isem):
        cid = lax.axis_index("c")
        sid = lax.axis_index("s")
        row0 = sid * ROWS_PER_TILE
        base = ((cid * NS + sid) if edge_split else sid) * nsb

        ng = nsb // G

        for k in range(chunks_per_core):
            if edge_split:
                uref = u.at[0]
            else:
                uref = u.at[cid * chunks_per_core + k]

            _zero_acc(acc, st2.at[0, 0], row0)
            plsc.subcore_barrier()

            def idx_load(g, m, sem=None):
                s_slc = src2d.at[pl.ds((base + g * G), G)]
                d_slc = dst2d.at[pl.ds((base + g * G), G)]
                if sem is None:
                    pltpu.sync_copy(s_slc, i3s.at[m])
                    pltpu.sync_copy(d_slc, i3d.at[m])
                else:
                    pltpu.async_copy(s_slc, i3s.at[m], sem)
                    pltpu.async_copy(d_slc, i3d.at[m], sem)

            def drain_gather(uref=uref):
                pltpu.make_async_copy(uref.at[i3s.at[0, 0]], st2.at[0, 0],
                                      gsem).wait()

            def drain_scatter():
                pltpu.make_async_copy(st2.at[0, 0], acc.at[i3d.at[0, 0]],
                                      ssem).wait()

            def drain_idx():
                pltpu.make_async_copy(src2d.at[pl.ds(0, G)],
                                      i3s.at[0], isem).wait()
                pltpu.make_async_copy(src2d.at[pl.ds(0, G)],
                                      i3d.at[0], isem).wait()

            idx_load(0, 0)
            idx_load(1, 1)
            for j in range(G):
                pltpu.async_copy(uref.at[i3s.at[0, j]], st2.at[0, j], gsem)

            def it(g, _, uref=uref):
                p = lax.rem(g, 2)
                pn = lax.rem(g + 1, 2)
                m = lax.rem(g, 3)
                mn = lax.rem(g + 1, 3)
                ml = lax.rem(g + 2, 3)

                for j in range(G):
                    drain_gather()

                @pl.when(g > 0)
                def _ws():
                    for j in range(G):
                        drain_scatter()

                @pl.when(g + 2 < ng)
                def _l():
                    idx_load(g + 2, ml, isem)

                for j in range(G):
                    pltpu.async_copy(st2.at[p, j], acc.at[i3d.at[m, j]],
                                     ssem, add=True)

                @pl.when(g + 1 < ng)
                def _g():
                    @pl.when(g > 0)
                    def _di():
                        drain_idx()
                    for j in range(G):
                        pltpu.async_copy(uref.at[i3s.at[mn, j]],
                                         st2.at[pn, j], gsem)

                return _

            lax.fori_loop(0, ng, it, None)
            for j in range(G):
                drain_scatter()
            plsc.subcore_barrier()

            oidx = cid if edge_split else cid * chunks_per_core + k
            pltpu.sync_copy(acc.at[pl.ds(row0, ROWS_PER_TILE)],
                            out.at[oidx, pl.ds(row0, ROWS_PER_TILE)])
            if k + 1 < chunks_per_core:
                plsc.subcore_barrier()

    return functools.partial(
        pl.kernel,
        out_type=jax.ShapeDtypeStruct((nout, NPAD, 16), jnp.float32),
        mesh=_mesh(),
        compiler_params=pltpu.CompilerParams(use_tc_tiling_on_sc=False),
        scratch_types=[
            pltpu.VMEM_SHARED((NPAD, 16), jnp.float32),
            pltpu.VMEM((3, G, SBE), jnp.int32),
            pltpu.VMEM((3, G, SBE), jnp.int32),
            pltpu.VMEM((2, G, SBE, 16), jnp.float32),
            pltpu.SemaphoreType.DMA,
            pltpu.SemaphoreType.DMA,
            pltpu.SemaphoreType.DMA,
        ],
    )(seg)


def _seg16(u, src2d, dst2d):
    return _make_segsum(1, True)(u, src2d, dst2d)


def _seg64(u, src2d, dst2d):
    return _make_segsum(4, False)(u, src2d, dst2d)


def _prologue_body(xp_ref, degp_ref, dinv_ref, u1_ref):
    i = pl.program_id(0)
    deg = degp_ref[0, :, :1] + degp_ref[1, :, :1] + 1.0
    rows = i * BLKN + lax.broadcasted_iota(jnp.int32, (BLKN, 1), 0)
    dinv = jnp.where(rows < N, lax.rsqrt(deg), 0.0)
    dinv_ref[...] = dinv
    u1_ref[0] = xp_ref[...] * dinv


def _layer_body(Sp_ref, u_ref, dinv_ref, W_ref, b_ref, out_ref, *, nsp, nch_in):
    dinv = dinv_ref[...]
    zs = []
    for c in range(nch_in):
        s = Sp_ref[0, c]
        for p in range(1, nsp):
            s = s + Sp_ref[p, c]
        zs.append((s + u_ref[c]) * dinv)
    zc = jnp.concatenate(zs, axis=1) if nch_in > 1 else zs[0]
    h = jnp.dot(zc, W_ref[...], preferred_element_type=jnp.float32) + b_ref[...]
    h = _leaky(h) * dinv
    for c in range(4):
        out_ref[c] = h[:, c * 16:(c + 1) * 16]


def _final_body(Sp_ref, u_ref, dinv_ref, W3_ref, b3_ref, Wfc_ref, bfc_ref,
                out_ref):
    dinv = dinv_ref[...]
    zs = [(Sp_ref[0, c] + u_ref[c]) * dinv for c in range(4)]
    zc = jnp.concatenate(zs, axis=1)
    h = _leaky(jnp.dot(zc, W3_ref[...], preferred_element_type=jnp.float32)
               + b3_ref[...])
    out_ref[...] = (jnp.dot(h, Wfc_ref[...], preferred_element_type=jnp.float32)
                    + bfc_ref[...])


def _row_spec(shape_prefix, blk_shape):
    nlead = len(shape_prefix)

    def imap(i):
        return tuple([0] * nlead) + (i, 0)

    return pl.BlockSpec(tuple(shape_prefix) + (blk_shape, 16), imap)


def _full_spec(shape):
    return pl.BlockSpec(shape, lambda i: tuple(0 for _ in shape))


_DINV_SPEC = pl.BlockSpec((BLKN, 1), lambda i: (i, 0))


def _prologue(xp, degp):
    return pl.pallas_call(
        _prologue_body,
        grid=(NBLK,),
        in_specs=[
            pl.BlockSpec((BLKN, 16), lambda i: (i, 0)),
            _row_spec((2,), BLKN),
        ],
        out_specs=[
            _DINV_SPEC,
            _row_spec((1,), BLKN),
        ],
        out_shape=[
            jax.ShapeDtypeStruct((NPAD, 1), jnp.float32),
            jax.ShapeDtypeStruct((1, NPAD, 16), jnp.float32),
        ],
    )(xp, degp)


def _layer(Sp, u, dinv, W, b, *, nsp, nch_in):
    body = functools.partial(_layer_body, nsp=nsp, nch_in=nch_in)
    return pl.pallas_call(
        body,
        grid=(NBLK,),
        in_specs=[
            _row_spec((nsp, nch_in), BLKN),
            _row_spec((nch_in,), BLKN),
            _DINV_SPEC,
            _full_spec((16 * nch_in, 64)),
            _full_spec((1, 64)),
        ],
        out_specs=_row_spec((4,), BLKN),
        out_shape=jax.ShapeDtypeStruct((4, NPAD, 16), jnp.float32),
    )(Sp, u, dinv, W, b)


def _final(Sp, u, dinv, W3, b3, Wfcp, bfcp):
    return pl.pallas_call(
        _final_body,
        grid=(NBLK,),
        in_specs=[
            _row_spec((1, 4), BLKN),
            _row_spec((4,), BLKN),
            _DINV_SPEC,
            _full_spec((64, 64)),
            _full_spec((1, 64)),
            _full_spec((64, 16)),
            _full_spec((1, 16)),
        ],
        out_specs=pl.BlockSpec((BLKN, 16), lambda i: (i, 0)),
        out_shape=jax.ShapeDtypeStruct((NPAD, 16), jnp.float32),
    )(Sp, u, dinv, W3, b3, Wfcp, bfcp)


def kernel(x, edge_index, W1, b1, W2, b2, W3, b3, Wfc, bfc):
    f32 = jnp.float32
    src = edge_index[0]
    dst = edge_index[1]
    pad = jnp.full((EPAD - E,), N, jnp.int32)
    src2d = jnp.concatenate([src, pad]).reshape(EBS, SBE)
    dst2d = jnp.concatenate([dst, pad]).reshape(EBS, SBE)
    xp = jnp.zeros((NPAD, 16), f32).at[:N, :10].set(x)

    W1p = jnp.zeros((16, 64), f32).at[:10].set(W1)
    Wfcp = jnp.zeros((64, 16), f32).at[:, :10].set(Wfc)
    b1r = b1.reshape(1, 64)
    b2r = b2.reshape(1, 64)
    b3r = b3.reshape(1, 64)
    bfcp = jnp.zeros((1, 16), f32).at[0, :10].set(bfc)

    degp = _deg_kernel(dst2d)
    dinv, u1 = _prologue(xp, degp)
    S1 = _seg16(u1, src2d, dst2d)
    u2 = _layer(S1[:, None], u1, dinv, W1p, b1r, nsp=2, nch_in=1)
    S2 = _seg64(u2, src2d, dst2d)
    u3 = _layer(S2[None], u2, dinv, W2, b2r, nsp=1, nch_in=4)
    S3 = _seg64(u3, src2d, dst2d)
    out = _final(S3[None], u3, dinv, W3, b3r, Wfcp, bfcp)
    return out[:N, :10]
